# baseline (speedup 1.0000x reference)
import functools

import jax
import jax.numpy as jnp
from jax import lax
from jax.experimental import pallas as pl
from jax.experimental.pallas import tpu as pltpu
from jax.experimental.pallas import tpu_sc as plsc

B = 16384
NC = 1
NS = 16
NW = NC * NS
BPW = B // NW
LANES = 16
NBLK = B // 128
BLK_PER_W = BPW // 128


def _make_gather_kernel():
    mesh = plsc.VectorSubcoreMesh(core_axis_name="c", subcore_axis_name="s",
                                  num_cores=NC)

    @functools.partial(
        pl.kernel,
        mesh=mesh,
        compiler_params=pltpu.CompilerParams(needs_layout_passes=False),
        out_type=[
            jax.ShapeDtypeStruct((NBLK, 2, 128), jnp.float32),
            jax.ShapeDtypeStruct((NBLK, 2, 128), jnp.float32),
            jax.ShapeDtypeStruct((B,), jnp.float32),
            jax.ShapeDtypeStruct((B,), jnp.float32),
        ],
        scratch_types=[
            pltpu.VMEM((2, 4), jnp.float32),
            pltpu.VMEM((2,), jnp.float32),
            pltpu.VMEM((LANES,), jnp.float32),
            pltpu.VMEM((BPW,), jnp.int32),
            pltpu.VMEM((BLK_PER_W, 2, 128), jnp.float32),
            pltpu.VMEM((BLK_PER_W, 2, 128), jnp.float32),
            pltpu.VMEM((BPW,), jnp.float32),
            pltpu.VMEM((BPW,), jnp.float32),
            pltpu.SemaphoreType.DMA,
            pltpu.SemaphoreType.DMA,
        ],
    )
    def gather_add(dirs_hbm, v_hbm, act_hbm,
                   out_hbm, half_hbm, ones_hbm, zero_hbm,
                   dirs_v, vv_v, tab_v, idx_v, out_v, half_v, ones_v, zero_v,
                   sem_in, sem_out):
        wid = lax.axis_index("s") * NC + lax.axis_index("c")
        base = wid * BPW
        h_act = pltpu.async_copy(act_hbm.at[pl.ds(base, BPW)], idx_v, sem_in)
        h_dirs = pltpu.async_copy(dirs_hbm, dirs_v, sem_in)
        h_v = pltpu.async_copy(v_hbm, vv_v, sem_in)
        halves = jnp.full((LANES,), 0.5, dtype=jnp.float32)
        ones = jnp.full((LANES,), 1.0, dtype=jnp.float32)
        zeros = jnp.zeros((LANES,), dtype=jnp.float32)

        @plsc.parallel_loop(0, BPW // LANES, unroll=2)
        def fill_body(k):
            t = k // 8
            off = (k - t * 8) * LANES
            half_v[t, 0, pl.ds(off, LANES)] = halves
            half_v[t, 1, pl.ds(off, LANES)] = halves
            ones_v[pl.ds(k * LANES, LANES)] = ones
            zero_v[pl.ds(k * LANES, LANES)] = zeros

        blk = pl.ds(wid * BLK_PER_W, BLK_PER_W)
        h_half = pltpu.async_copy(half_v, half_hbm.at[blk], sem_out)
        h_ones = pltpu.async_copy(ones_v, ones_hbm.at[pl.ds(base, BPW)], sem_out)
        h_zero = pltpu.async_copy(zero_v, zero_hbm.at[pl.ds(base, BPW)], sem_out)
        h_act.wait()
        h_dirs.wait()
        h_v.wait()
        lane = lax.iota(jnp.int32, LANES)
        row = lane // 2
        col = lane - row * 2
        tab_v[...] = (plsc.load_gather(dirs_v, [col, row])
                      + plsc.load_gather(vv_v, [col]))

        @plsc.parallel_loop(0, BPW // LANES, unroll=2)
        def body(k):
            t = k // 8
            off = (k - t * 8) * LANES
            a = idx_v[pl.ds(k * LANES, LANES)]
            i0 = a * 2
            g0 = plsc.load_gather(tab_v, [i0])
            g1 = plsc.load_gather(tab_v, [i0 + 1])
            out_v[t, 0, pl.ds(off, LANES)] = g0
            out_v[t, 1, pl.ds(off, LANES)] = g1

        h_out = pltpu.async_copy(out_v, out_hbm.at[blk], sem_out)
        h_half.wait()
        h_ones.wait()
        h_zero.wait()
        h_out.wait()

    return gather_add


_gather_add = _make_gather_kernel()


def kernel(state, context, action, v, dirs):
    act_flat = action.reshape(B)
    out_blocks, half2, ones1, zero1 = _gather_add(dirs.T, v, act_flat)
    delta = out_blocks.transpose(0, 2, 1).reshape(B, 1, 2)
    halves = half2.transpose(0, 2, 1).reshape(B, 1, 2)
    weights = ones1.reshape(B, 1)
    resid = zero1
    return ((delta, halves), weights, resid)


# speedup vs baseline: 1.0058x; 1.0058x over previous
import functools

import jax
import jax.numpy as jnp
from jax import lax
from jax.experimental import pallas as pl
from jax.experimental.pallas import tpu as pltpu
from jax.experimental.pallas import tpu_sc as plsc

B = 16384
NC = 1
NS = 16
NW = NC * NS
BPW = B // NW
LANES = 16
NBLK = B // 128
BLK_PER_W = BPW // 128


def _make_gather_kernel():
    mesh = plsc.VectorSubcoreMesh(core_axis_name="c", subcore_axis_name="s",
                                  num_cores=NC)

    @functools.partial(
        pl.kernel,
        mesh=mesh,
        compiler_params=pltpu.CompilerParams(needs_layout_passes=False),
        out_type=[
            jax.ShapeDtypeStruct((NBLK, 2, 128), jnp.float32),
            jax.ShapeDtypeStruct((NBLK, 2, 128), jnp.float32),
            jax.ShapeDtypeStruct((B,), jnp.float32),
            jax.ShapeDtypeStruct((B,), jnp.float32),
        ],
        scratch_types=[
            pltpu.VMEM((2, 4), jnp.float32),
            pltpu.VMEM((2,), jnp.float32),
            pltpu.VMEM((LANES,), jnp.float32),
            pltpu.VMEM((BPW,), jnp.int32),
            pltpu.VMEM((BLK_PER_W, 2, 128), jnp.float32),
            pltpu.VMEM((BLK_PER_W, 2, 128), jnp.float32),
            pltpu.VMEM((BPW,), jnp.float32),
            pltpu.VMEM((BPW,), jnp.float32),
            pltpu.SemaphoreType.DMA,
            pltpu.SemaphoreType.DMA,
        ],
    )
    def gather_add(dirs_hbm, v_hbm, act_hbm,
                   out_hbm, half_hbm, ones_hbm, zero_hbm,
                   dirs_v, vv_v, tab_v, idx_v, out_v, half_v, ones_v, zero_v,
                   sem_in, sem_out):
        wid = lax.axis_index("s") * NC + lax.axis_index("c")
        base = wid * BPW
        h_act = pltpu.async_copy(act_hbm.at[pl.ds(base, BPW)], idx_v, sem_in)
        h_dirs = pltpu.async_copy(dirs_hbm, dirs_v, sem_in)
        h_v = pltpu.async_copy(v_hbm, vv_v, sem_in)
        halves = jnp.full((LANES,), 0.5, dtype=jnp.float32)
        ones = jnp.full((LANES,), 1.0, dtype=jnp.float32)
        zeros = jnp.zeros((LANES,), dtype=jnp.float32)

        h_act.wait()
        h_dirs.wait()
        h_v.wait()
        lane = lax.iota(jnp.int32, LANES)
        row = lane // 2
        col = lane - row * 2
        tab_v[...] = (plsc.load_gather(dirs_v, [col, row])
                      + plsc.load_gather(vv_v, [col]))

        @plsc.parallel_loop(0, BPW // LANES, unroll=2)
        def body(k):
            t = k // 8
            off = (k - t * 8) * LANES
            a = idx_v[pl.ds(k * LANES, LANES)]
            i0 = a * 2
            g0 = plsc.load_gather(tab_v, [i0])
            g1 = plsc.load_gather(tab_v, [i0 + 1])
            out_v[t, 0, pl.ds(off, LANES)] = g0
            out_v[t, 1, pl.ds(off, LANES)] = g1
            half_v[t, 0, pl.ds(off, LANES)] = halves
            half_v[t, 1, pl.ds(off, LANES)] = halves
            ones_v[pl.ds(k * LANES, LANES)] = ones
            zero_v[pl.ds(k * LANES, LANES)] = zeros

        blk = pl.ds(wid * BLK_PER_W, BLK_PER_W)
        h_half = pltpu.async_copy(half_v, half_hbm.at[blk], sem_out)
        h_ones = pltpu.async_copy(ones_v, ones_hbm.at[pl.ds(base, BPW)], sem_out)
        h_zero = pltpu.async_copy(zero_v, zero_hbm.at[pl.ds(base, BPW)], sem_out)
        h_out = pltpu.async_copy(out_v, out_hbm.at[blk], sem_out)
        h_half.wait()
        h_ones.wait()
        h_zero.wait()
        h_out.wait()

    return gather_add


_gather_add = _make_gather_kernel()


def kernel(state, context, action, v, dirs):
    act_flat = action.reshape(B)
    out_blocks, half2, ones1, zero1 = _gather_add(dirs.T, v, act_flat)
    delta = out_blocks.transpose(0, 2, 1).reshape(B, 1, 2)
    halves = half2.transpose(0, 2, 1).reshape(B, 1, 2)
    weights = ones1.reshape(B, 1)
    resid = zero1
    return ((delta, halves), weights, resid)
